# baseline (device time: 353956 ns/iter reference)
import jax
import jax.numpy as jnp
from jax import lax
from jax.experimental import pallas as pl
from jax.experimental.pallas import tpu as pltpu

N_DEV = 4
S_PER = 1024
S = N_DEV * S_PER
D = 1024
H = 8
DH = 128
SCALE = 0.08838834764831843
BF = jnp.bfloat16

KV_BLK = 512
Q_BLK = 256


def _body(x_ref, wq_ref, wqr_ref, wk_ref, wkr_ref, wv_ref, wo_ref,
          cos_ref, sin_ref, out_ref,
          xf_ref, k_ref, v_ref, rsb_ref,
          ag_send, ag_recv, rs_send, rs_recv):
    me = lax.axis_index("i")
    left = lax.rem(me + N_DEV - 1, N_DEV)
    right = lax.rem(me + 1, N_DEV)

    barrier = pltpu.get_barrier_semaphore()
    for nbr in (left, right):
        pl.semaphore_signal(barrier, inc=1, device_id=(nbr,),
                            device_id_type=pl.DeviceIdType.MESH)
    pl.semaphore_wait(barrier, 2)

    def tile_cs(rows):
        cosb = jnp.concatenate([cos_ref[rows, :]] * H, axis=1)
        sinb = jnp.concatenate([sin_ref[rows, :]] * H, axis=1)
        return cosb, sinb

    def kv_chunk(c):
        def blk(i, carry):
            rows = pl.ds(c * S_PER + i * KV_BLK, KV_BLK)
            xb = xf_ref[rows, :]
            cosb, sinb = tile_cs(rows)
            k0 = jnp.dot(xb, wk_ref[...], preferred_element_type=jnp.float32)
            kr = jnp.dot(xb, wkr_ref[...], preferred_element_type=jnp.float32)
            k_ref[rows, :] = (k0 * cosb + kr * sinb).astype(BF)
            v_ref[rows, :] = jnp.dot(
                xb, wv_ref[...], preferred_element_type=jnp.float32).astype(BF)
            return carry
        lax.fori_loop(0, S_PER // KV_BLK, blk, 0)

    def q_chunk(c):
        def blk(i, carry):
            rows = pl.ds(c * S_PER + i * KV_BLK, KV_BLK)
            xb = xf_ref[rows, :]
            cosb, sinb = tile_cs(rows)
            q0 = jnp.dot(xb, wq_ref[...], preferred_element_type=jnp.float32)
            qr = jnp.dot(xb, wqr_ref[...], preferred_element_type=jnp.float32)
            xf_ref[rows, :] = (q0 * cosb + qr * sinb).astype(BF)
            return carry
        lax.fori_loop(0, S_PER // KV_BLK, blk, 0)

    xf_ref[pl.ds(me * S_PER, S_PER), :] = x_ref[...]
    for h in range(N_DEV - 1):
        c = lax.rem(me - h + N_DEV, N_DEV)
        rdma = pltpu.make_async_remote_copy(
            src_ref=xf_ref.at[pl.ds(c * S_PER, S_PER), :],
            dst_ref=xf_ref.at[pl.ds(c * S_PER, S_PER), :],
            send_sem=ag_send.at[h],
            recv_sem=ag_recv.at[h],
            device_id=(right,),
            device_id_type=pl.DeviceIdType.MESH,
        )
        rdma.start()
        if h > 0:
            q_chunk(lax.rem(me - h + 1 + N_DEV, N_DEV))
        kv_chunk(c)
        rdma.wait()
    q_chunk(lax.rem(me - 2 + N_DEV, N_DEV))
    kv_chunk(lax.rem(me - 3 + N_DEV, N_DEV))
    q_chunk(lax.rem(me - 3 + N_DEV, N_DEV))

    def attn_chunk(c):
        def blk(i, carry):
            rows = pl.ds(c * S_PER + i * Q_BLK, Q_BLK)
            q = xf_ref[rows, :]
            ctx_heads = []
            for hh in range(H):
                cols = slice(hh * DH, (hh + 1) * DH)
                s = lax.dot_general(
                    q[:, cols], k_ref[:, cols], (((1,), (1,)), ((), ())),
                    preferred_element_type=jnp.float32)
                w = jnp.exp2(s.astype(BF))
                denom = jnp.sum(w, axis=-1, keepdims=True,
                                dtype=jnp.float32)
                ctx = jnp.dot(w, v_ref[:, cols],
                              preferred_element_type=jnp.float32)
                ctx_heads.append((ctx / denom).astype(BF))
            ctx = jnp.concatenate(ctx_heads, axis=1)
            p = jnp.dot(ctx, wo_ref[...], preferred_element_type=jnp.float32)
            xf_ref[rows, :] = p.astype(BF)
            return carry
        lax.fori_loop(0, S_PER // Q_BLK, blk, 0)

    rdma_prev = None
    for h in range(N_DEV):
        c = lax.rem(me + N_DEV - 1 - h, N_DEV)
        attn_chunk(c)
        if rdma_prev is not None:
            rdma_prev.wait()
            acc = (xf_ref[pl.ds(c * S_PER, S_PER), :].astype(jnp.float32)
                   + rsb_ref[h - 1].astype(jnp.float32))
            if h < N_DEV - 1:
                xf_ref[pl.ds(c * S_PER, S_PER), :] = acc.astype(BF)
            else:
                out_ref[...] = acc
        if h < N_DEV - 1:
            rdma_prev = pltpu.make_async_remote_copy(
                src_ref=xf_ref.at[pl.ds(c * S_PER, S_PER), :],
                dst_ref=rsb_ref.at[h],
                send_sem=rs_send.at[h],
                recv_sem=rs_recv.at[h],
                device_id=(right,),
                device_id_type=pl.DeviceIdType.MESH,
            )
            rdma_prev.start()


def kernel(x, Wq, Wk, Wv, Wo):
    x2 = x.reshape(S_PER, D).astype(BF)

    def rot_w(w):
        w2 = w.reshape(D, D // 2, 2)
        return jnp.stack([-w2[..., 1], w2[..., 0]], axis=-1).reshape(D, D)

    qscale = SCALE * 1.4426950408889634
    wq = (Wq * qscale).astype(BF)
    wk = Wk.astype(BF)
    wv = Wv.astype(BF)
    wo = Wo.astype(BF)
    wqr = (rot_w(Wq) * qscale).astype(BF)
    wkr = rot_w(Wk).astype(BF)

    inv = 1.0 / (10000.0 ** (jnp.arange(0, DH, 2, dtype=jnp.float32) / DH))
    pos = jnp.arange(S, dtype=jnp.float32)[:, None] * inv[None, :]
    cos = jnp.repeat(jnp.cos(pos), 2, axis=-1)
    sin = jnp.repeat(jnp.sin(pos), 2, axis=-1)

    out = pl.pallas_call(
        _body,
        out_shape=jax.ShapeDtypeStruct((S_PER, D), jnp.float32),
        in_specs=[pl.BlockSpec(memory_space=pltpu.VMEM)] * 9,
        out_specs=pl.BlockSpec(memory_space=pltpu.VMEM),
        scratch_shapes=[
            pltpu.VMEM((S, D), BF),
            pltpu.VMEM((S, D), BF),
            pltpu.VMEM((S, D), BF),
            pltpu.VMEM((N_DEV - 1, S_PER, D), BF),
            pltpu.SemaphoreType.DMA((N_DEV - 1,)),
            pltpu.SemaphoreType.DMA((N_DEV - 1,)),
            pltpu.SemaphoreType.DMA((N_DEV - 1,)),
            pltpu.SemaphoreType.DMA((N_DEV - 1,)),
        ],
        compiler_params=pltpu.CompilerParams(
            collective_id=0, vmem_limit_bytes=63 * 1024 * 1024),
    )(x2, wq, wqr, wk, wkr, wv, wo, cos, sin)
    return out.reshape(1, S_PER, D).astype(jnp.float32)


# device time: 249353 ns/iter; 1.4195x vs baseline; 1.4195x over previous
import jax
import jax.numpy as jnp
from jax import lax
from jax.experimental import pallas as pl
from jax.experimental.pallas import tpu as pltpu

N_DEV = 4
S_PER = 1024
S = N_DEV * S_PER
D = 1024
H = 8
DH = 128
SCALE = 0.08838834764831843
BF = jnp.bfloat16

KV_BLK = 512
Q_BLK = 256


def _body(x_ref, wq_ref, wqr_ref, wk_ref, wkr_ref, wv_ref, wo_ref,
          cos_ref, sin_ref, out_ref,
          xf_ref, k_ref, v_ref, rsb_ref,
          ag_send, ag_recv, rs_send, rs_recv):
    me = lax.axis_index("i")
    left = lax.rem(me + N_DEV - 1, N_DEV)
    right = lax.rem(me + 1, N_DEV)

    barrier = pltpu.get_barrier_semaphore()
    for nbr in (left, right):
        pl.semaphore_signal(barrier, inc=1, device_id=(nbr,),
                            device_id_type=pl.DeviceIdType.MESH)
    pl.semaphore_wait(barrier, 2)

    def tile_cs(rows):
        cosb = jnp.concatenate([cos_ref[rows, :]] * H, axis=1)
        sinb = jnp.concatenate([sin_ref[rows, :]] * H, axis=1)
        return cosb, sinb

    def kv_chunk(c):
        def blk(i, carry):
            rows = pl.ds(c * S_PER + i * KV_BLK, KV_BLK)
            xb = xf_ref[rows, :]
            cosb, sinb = tile_cs(rows)
            k0 = jnp.dot(xb, wk_ref[...], preferred_element_type=jnp.float32)
            kr = jnp.dot(xb, wkr_ref[...], preferred_element_type=jnp.float32)
            k_ref[rows, :] = (k0 * cosb + kr * sinb).astype(BF)
            v_ref[rows, :] = jnp.dot(
                xb, wv_ref[...], preferred_element_type=jnp.float32).astype(BF)
            return carry
        lax.fori_loop(0, S_PER // KV_BLK, blk, 0)

    def q_chunk(c):
        def blk(i, carry):
            rows = pl.ds(c * S_PER + i * KV_BLK, KV_BLK)
            xb = xf_ref[rows, :]
            cosb, sinb = tile_cs(rows)
            q0 = jnp.dot(xb, wq_ref[...], preferred_element_type=jnp.float32)
            qr = jnp.dot(xb, wqr_ref[...], preferred_element_type=jnp.float32)
            xf_ref[rows, :] = (q0 * cosb + qr * sinb).astype(BF)
            return carry
        lax.fori_loop(0, S_PER // KV_BLK, blk, 0)

    xf_ref[pl.ds(me * S_PER, S_PER), :] = x_ref[...]
    for h in range(N_DEV - 1):
        c = lax.rem(me - h + N_DEV, N_DEV)
        rdma = pltpu.make_async_remote_copy(
            src_ref=xf_ref.at[pl.ds(c * S_PER, S_PER), :],
            dst_ref=xf_ref.at[pl.ds(c * S_PER, S_PER), :],
            send_sem=ag_send.at[h],
            recv_sem=ag_recv.at[h],
            device_id=(right,),
            device_id_type=pl.DeviceIdType.MESH,
        )
        rdma.start()
        if h > 0:
            q_chunk(lax.rem(me - h + 1 + N_DEV, N_DEV))
        kv_chunk(c)
        rdma.wait()
    q_chunk(lax.rem(me - 2 + N_DEV, N_DEV))
    kv_chunk(lax.rem(me - 3 + N_DEV, N_DEV))
    q_chunk(lax.rem(me - 3 + N_DEV, N_DEV))

    def attn_chunk(c):
        def blk(i, carry):
            rows = pl.ds(c * S_PER + i * Q_BLK, Q_BLK)
            q = xf_ref[rows, :]
            ctx_heads = []
            for hh in range(H):
                cols = slice(hh * DH, (hh + 1) * DH)
                s = lax.dot_general(
                    q[:, cols], k_ref[:, cols], (((1,), (1,)), ((), ())),
                    preferred_element_type=jnp.float32)
                w = jnp.exp2(s)
                denom = jnp.sum(w, axis=-1, keepdims=True)
                ctx = jnp.dot(w.astype(BF), v_ref[:, cols],
                              preferred_element_type=jnp.float32)
                ctx_heads.append((ctx / denom).astype(BF))
            ctx = jnp.concatenate(ctx_heads, axis=1)
            p = jnp.dot(ctx, wo_ref[...], preferred_element_type=jnp.float32)
            xf_ref[rows, :] = p.astype(BF)
            return carry
        lax.fori_loop(0, S_PER // Q_BLK, blk, 0)

    rdma_prev = None
    for h in range(N_DEV):
        c = lax.rem(me + N_DEV - 1 - h, N_DEV)
        attn_chunk(c)
        if rdma_prev is not None:
            rdma_prev.wait()
            acc = (xf_ref[pl.ds(c * S_PER, S_PER), :].astype(jnp.float32)
                   + rsb_ref[h - 1].astype(jnp.float32))
            if h < N_DEV - 1:
                xf_ref[pl.ds(c * S_PER, S_PER), :] = acc.astype(BF)
            else:
                out_ref[...] = acc
        if h < N_DEV - 1:
            rdma_prev = pltpu.make_async_remote_copy(
                src_ref=xf_ref.at[pl.ds(c * S_PER, S_PER), :],
                dst_ref=rsb_ref.at[h],
                send_sem=rs_send.at[h],
                recv_sem=rs_recv.at[h],
                device_id=(right,),
                device_id_type=pl.DeviceIdType.MESH,
            )
            rdma_prev.start()


def kernel(x, Wq, Wk, Wv, Wo):
    x2 = x.reshape(S_PER, D).astype(BF)

    def rot_w(w):
        w2 = w.reshape(D, D // 2, 2)
        return jnp.stack([-w2[..., 1], w2[..., 0]], axis=-1).reshape(D, D)

    qscale = SCALE * 1.4426950408889634
    wq = (Wq * qscale).astype(BF)
    wk = Wk.astype(BF)
    wv = Wv.astype(BF)
    wo = Wo.astype(BF)
    wqr = (rot_w(Wq) * qscale).astype(BF)
    wkr = rot_w(Wk).astype(BF)

    inv = 1.0 / (10000.0 ** (jnp.arange(0, DH, 2, dtype=jnp.float32) / DH))
    pos = jnp.arange(S, dtype=jnp.float32)[:, None] * inv[None, :]
    cos = jnp.repeat(jnp.cos(pos), 2, axis=-1)
    sin = jnp.repeat(jnp.sin(pos), 2, axis=-1)

    out = pl.pallas_call(
        _body,
        out_shape=jax.ShapeDtypeStruct((S_PER, D), jnp.float32),
        in_specs=[pl.BlockSpec(memory_space=pltpu.VMEM)] * 9,
        out_specs=pl.BlockSpec(memory_space=pltpu.VMEM),
        scratch_shapes=[
            pltpu.VMEM((S, D), BF),
            pltpu.VMEM((S, D), BF),
            pltpu.VMEM((S, D), BF),
            pltpu.VMEM((N_DEV - 1, S_PER, D), BF),
            pltpu.SemaphoreType.DMA((N_DEV - 1,)),
            pltpu.SemaphoreType.DMA((N_DEV - 1,)),
            pltpu.SemaphoreType.DMA((N_DEV - 1,)),
            pltpu.SemaphoreType.DMA((N_DEV - 1,)),
        ],
        compiler_params=pltpu.CompilerParams(
            collective_id=0, vmem_limit_bytes=63 * 1024 * 1024),
    )(x2, wq, wqr, wk, wkr, wv, wo, cos, sin)
    return out.reshape(1, S_PER, D).astype(jnp.float32)
